# Initial kernel scaffold; baseline (speedup 1.0000x reference)
import jax
import jax.numpy as jnp
from jax.experimental import pallas as pl


def kernel(x, edge_index, params):
    raise NotImplementedError("write your pallas kernel here")


# speedup vs baseline: 2.1360x; 2.1360x over previous
import functools

import jax
import jax.numpy as jnp
from jax import lax
from jax.experimental import pallas as pl
from jax.experimental.pallas import tpu as pltpu
from jax.experimental.pallas import tpu_sc as plsc

_N = 10000
_E = 320000
_C = 128
_NEG = 0.2
_BN_EPS = 1e-5
_NPAD = 10240
_BLK = 128
_NBLK = _E // _BLK
_NW = 32
_RPT = _NPAD // 16


def _sc_edge_call(xl, xr, src_r, dst_r, att, zero_nc, zero_np):
    mesh = plsc.VectorSubcoreMesh(core_axis_name="c", subcore_axis_name="s")

    @functools.partial(
        pl.kernel,
        out_type=[
            jax.ShapeDtypeStruct((2, _NPAD, _C), jnp.float32),
            jax.ShapeDtypeStruct((2, _NPAD), jnp.float32),
        ],
        mesh=mesh,
        compiler_params=pltpu.CompilerParams(needs_layout_passes=False),
        scratch_types=[
            pltpu.VMEM((1, _BLK), jnp.int32),
            pltpu.VMEM((1, _BLK), jnp.int32),
            pltpu.VMEM((_BLK, _C), jnp.float32),
            pltpu.VMEM((_BLK, _C), jnp.float32),
            pltpu.VMEM((_BLK,), jnp.float32),
            pltpu.VMEM((_C,), jnp.float32),
            pltpu.VMEM_SHARED((_NPAD, _C), jnp.float32),
            pltpu.VMEM_SHARED((_NPAD,), jnp.float32),
            pltpu.SemaphoreType.DMA,
            pltpu.SemaphoreType.DMA,
        ],
    )
    def k(xl_h, xr_h, src_h, dst_h, att_h, znc_h, znp_h, s_out, den_out,
          src_v, dst_v, xlr, xrr, ex_v, att_v, s_sh, den_sh, sem1, sem2):
        cid = lax.axis_index("c")
        sid = lax.axis_index("s")
        wid = cid * 16 + sid

        r0 = sid * _RPT
        pltpu.sync_copy(znc_h.at[pl.ds(r0, _RPT)], s_sh.at[pl.ds(r0, _RPT)])

        @pl.when(sid == 0)
        def _():
            pltpu.sync_copy(znp_h, den_sh)

        pltpu.sync_copy(att_h, att_v)
        plsc.subcore_barrier()

        niters = jnp.where(wid < _NBLK % _NW, _NBLK // _NW + 1, _NBLK // _NW)

        def body(i, carry):
            b = wid + i * _NW
            pltpu.sync_copy(src_h.at[pl.ds(b, 1)], src_v)
            pltpu.sync_copy(dst_h.at[pl.ds(b, 1)], dst_v)
            cp1 = pltpu.async_copy(xl_h.at[src_v.at[0]], xlr, sem1)
            cp2 = pltpu.async_copy(xr_h.at[dst_v.at[0]], xrr, sem2)
            cp1.wait()
            cp2.wait()
            for g in range(_BLK // 16):
                ev = lax.iota(jnp.int32, 16) + (g * 16)

                def cstep(s, acc):
                    cbase = s * 16
                    att_vec = att_v[pl.ds(cbase, 16)]
                    for j in range(16):
                        cvec = jnp.full((16,), j, jnp.int32) + cbase
                        gl = plsc.load_gather(xlr, [ev, cvec])
                        gr = plsc.load_gather(xrr, [ev, cvec])
                        v = gl + gr
                        lr = jnp.maximum(v, _NEG * v)
                        acc = acc + lr * att_vec[j]
                    return acc

                acc = lax.fori_loop(0, _C // 16, cstep,
                                    jnp.zeros((16,), jnp.float32))
                ex16 = jnp.exp(acc)
                ex_v[pl.ds(g * 16, 16)] = ex16

                def sstep(s, carry2):
                    cbase = s * 16
                    for j in range(16):
                        cvec = jnp.full((16,), j, jnp.int32) + cbase
                        gl = plsc.load_gather(xlr, [ev, cvec])
                        plsc.store_scatter(xlr, [ev, cvec], gl * ex16)
                    return carry2

                lax.fori_loop(0, _C // 16, sstep, 0)
            pltpu.sync_copy(ex_v, den_sh.at[dst_v.at[0]], add=True)
            pltpu.sync_copy(xlr, s_sh.at[dst_v.at[0]], add=True)
            return carry

        lax.fori_loop(0, niters, body, 0)

        plsc.subcore_barrier()
        pltpu.sync_copy(s_sh.at[pl.ds(r0, _RPT)],
                        s_out.at[cid, pl.ds(r0, _RPT)])

        @pl.when(sid == 0)
        def _():
            pltpu.sync_copy(den_sh, den_out.at[cid])

    return k(xl, xr, src_r, dst_r, att, zero_nc, zero_np)


def _pre_body(x_r, wl_r, bl_r, wr_r, br_r, xl_o, xr_o):
    h = x_r[...]
    xl_o[...] = jnp.dot(h, wl_r[...],
                        preferred_element_type=jnp.float32) + bl_r[...]
    xr_o[...] = jnp.dot(h, wr_r[...],
                        preferred_element_type=jnp.float32) + br_r[...]


def _combine(s_r, den_r, bias_r, wlin_r, blin_r, gamma_r, beta_r, do_relu):
    s3 = s_r[...]
    s = s3[0, :_N] + s3[1, :_N]
    out = s / (den_r[...] + 1e-16) + bias_r[...]
    t = out + jnp.dot(out, wlin_r[...],
                      preferred_element_type=jnp.float32) + blin_r[...]
    mean = jnp.mean(t, axis=0, keepdims=True)
    var = jnp.mean((t - mean) ** 2, axis=0, keepdims=True)
    hh = (t - mean) * lax.rsqrt(var + _BN_EPS) * gamma_r[...] + beta_r[...]
    if do_relu:
        hh = jnp.maximum(hh, 0.0)
    return hh


def _mid_body(s_r, den_r, bias_r, wlin_r, blin_r, gamma_r, beta_r,
              wl_r, bl_r, wr_r, br_r, xl_o, xr_o):
    hh = _combine(s_r, den_r, bias_r, wlin_r, blin_r, gamma_r, beta_r, True)
    xl_o[...] = jnp.dot(hh, wl_r[...],
                        preferred_element_type=jnp.float32) + bl_r[...]
    xr_o[...] = jnp.dot(hh, wr_r[...],
                        preferred_element_type=jnp.float32) + br_r[...]


def _fin_body(s_r, den_r, bias_r, wlin_r, blin_r, gamma_r, beta_r, h_o):
    h_o[...] = _combine(s_r, den_r, bias_r, wlin_r, blin_r, gamma_r, beta_r,
                        False)


def kernel(x, edge_index, params):
    src_r = edge_index[0].reshape(_NBLK, _BLK)
    dst_r = edge_index[1].reshape(_NBLK, _BLK)
    zero_nc = jnp.zeros((_NPAD, _C), jnp.float32)
    zero_np = jnp.zeros((_NPAD,), jnp.float32)
    p0, p1 = params

    nc = jax.ShapeDtypeStruct((_N, _C), jnp.float32)

    xl, xr = pl.pallas_call(
        _pre_body, out_shape=[nc, nc],
    )(x, p0['Wl'], p0['bl'].reshape(1, _C), p0['Wr'], p0['br'].reshape(1, _C))

    s2, den2 = _sc_edge_call(xl, xr, src_r, dst_r, p0['att'],
                             zero_nc, zero_np)
    den = (den2[0, :_N] + den2[1, :_N]).reshape(_N, 1)

    xl2, xr2 = pl.pallas_call(
        _mid_body, out_shape=[nc, nc],
    )(s2, den, p0['bias'].reshape(1, _C), p0['Wlin'],
      p0['blin'].reshape(1, _C), p0['gamma'].reshape(1, _C),
      p0['beta'].reshape(1, _C), p1['Wl'], p1['bl'].reshape(1, _C),
      p1['Wr'], p1['br'].reshape(1, _C))

    s2b, den2b = _sc_edge_call(xl2, xr2, src_r, dst_r, p1['att'],
                               zero_nc, zero_np)
    denb = (den2b[0, :_N] + den2b[1, :_N]).reshape(_N, 1)

    h = pl.pallas_call(
        _fin_body, out_shape=nc,
    )(s2b, denb, p1['bias'].reshape(1, _C), p1['Wlin'],
      p1['blin'].reshape(1, _C), p1['gamma'].reshape(1, _C),
      p1['beta'].reshape(1, _C))

    return h


# speedup vs baseline: 2.2850x; 1.0697x over previous
import functools

import jax
import jax.numpy as jnp
from jax import lax
from jax.experimental import pallas as pl
from jax.experimental.pallas import tpu as pltpu
from jax.experimental.pallas import tpu_sc as plsc

_N = 10000
_E = 320000
_C = 128
_NEG = 0.2
_BN_EPS = 1e-5
_NPAD = 10112
_BLK = 64
_NW = 32
_CHB = 9
_NCH = 18
_NBW = _NCH * _CHB
_NB = _NW * _NBW
_E2 = _NB * _BLK
_RPT = _NPAD // 16


def _sc_edge_call(xl, xr, src_r, dst_r, att, zero_nc, zero_np):
    mesh = plsc.VectorSubcoreMesh(core_axis_name="c", subcore_axis_name="s")

    @functools.partial(
        pl.kernel,
        out_type=[
            jax.ShapeDtypeStruct((2, _NPAD, _C), jnp.float32),
            jax.ShapeDtypeStruct((2, _NPAD), jnp.float32),
        ],
        mesh=mesh,
        compiler_params=pltpu.CompilerParams(needs_layout_passes=False),
        scratch_types=[
            [pltpu.VMEM((_CHB * _BLK,), jnp.int32)] * 2,
            [pltpu.VMEM((_CHB, _BLK), jnp.int32)] * 2,
            [pltpu.VMEM((_BLK, _C), jnp.float32)] * 3,
            [pltpu.VMEM((_BLK, _C), jnp.float32)] * 2,
            [pltpu.VMEM((_BLK,), jnp.float32)] * 2,
            pltpu.VMEM((_C,), jnp.float32),
            pltpu.VMEM_SHARED((_NPAD, _C), jnp.float32),
            pltpu.VMEM_SHARED((_NPAD,), jnp.float32),
            [pltpu.SemaphoreType.DMA] * 3,
            [pltpu.SemaphoreType.DMA] * 2,
            [pltpu.SemaphoreType.DMA] * 3,
            [pltpu.SemaphoreType.DMA] * 2,
        ],
    )
    def k(xl_h, xr_h, src_h, dst_h, att_h, znc_h, znp_h, s_out, den_out,
          srcb, dstb, xlr, xrr, ex_v, att_v, s_sh, den_sh,
          gls, grs, scs, ixs):
        cid = lax.axis_index("c")
        sid = lax.axis_index("s")
        wid = cid * 16 + sid

        r0 = sid * _RPT
        pltpu.sync_copy(znc_h.at[pl.ds(r0, _RPT)], s_sh.at[pl.ds(r0, _RPT)])

        @pl.when(sid == 0)
        def _():
            pltpu.sync_copy(znp_h, den_sh)

        pltpu.sync_copy(att_h, att_v)
        pltpu.sync_copy(src_h.at[wid, 0], srcb[0])
        pltpu.sync_copy(dst_h.at[wid, 0], dstb[0])
        plsc.subcore_barrier()

        def g_xl(bp, bi, d):
            return pltpu.make_async_copy(
                xl_h.at[srcb[bp].at[pl.ds(bi * _BLK, _BLK)]], xlr[d], gls[d])

        def g_xr(bp, bi, d):
            return pltpu.make_async_copy(
                xr_h.at[dstb[bp].at[bi]], xrr[d], grs[d])

        def sc_pair(bp, bi, d, de):
            return (pltpu.make_async_copy(ex_v[de],
                                          den_sh.at[dstb[bp].at[bi]],
                                          scs[d]),
                    pltpu.make_async_copy(xlr[d], s_sh.at[dstb[bp].at[bi]],
                                          scs[d]))

        g_xl(0, 0, 0).start()
        g_xr(0, 0, 0).start()
        g_xl(0, 1, 1).start()
        g_xr(0, 1, 1).start()

        def compute(d3, d2):
            def group(g, carry):
                ev = lax.iota(jnp.int32, 16) + g * 16

                def cstep(s, acc):
                    cbase = pl.multiple_of(s * 16, 16)
                    att_vec = att_v[pl.ds(cbase, 16)]
                    for j in range(16):
                        cvec = jnp.full((16,), j, jnp.int32) + cbase
                        vl = plsc.load_gather(xlr[d3], [ev, cvec])
                        vr = plsc.load_gather(xrr[d2], [ev, cvec])
                        v = vl + vr
                        lr = jnp.maximum(v, _NEG * v)
                        acc = acc + lr * att_vec[j]
                    return acc

                acc = lax.fori_loop(0, _C // 16, cstep,
                                    jnp.zeros((16,), jnp.float32))
                ex16 = jnp.exp(acc)
                ex_v[d2][pl.ds(pl.multiple_of(g * 16, 16), 16)] = ex16

                def sstep(s, carry2):
                    cbase = pl.multiple_of(s * 16, 16)
                    for j in range(16):
                        cvec = jnp.full((16,), j, jnp.int32) + cbase
                        vl = plsc.load_gather(xlr[d3], [ev, cvec])
                        plsc.store_scatter(xlr[d3], [ev, cvec], vl * ex16)
                    return carry2

                lax.fori_loop(0, _C // 16, sstep, 0)
                return carry

            lax.fori_loop(0, _BLK // 16, group, 0)

        def body(ii, carry):
            for p in range(2):
                kk = ii * 2 + p
                pnext = (p + 1) % 2

                for bi in range(_CHB):
                    b = kk * _CHB + bi
                    d3 = bi % 3
                    d2 = (p + bi) % 2
                    g_xl(p, bi, d3).wait()
                    g_xr(p, bi, d2).wait()
                    compute(d3, d2)
                    e1, e2 = sc_pair(p, bi, d3, d2)
                    e1.start(add=True)
                    e2.start(add=True)

                    d3n = (bi + 2) % 3

                    @pl.when(b >= 1)
                    def _():
                        if bi >= 1:
                            w1, w2 = sc_pair(p, bi - 1, d3n, (p + bi - 1) % 2)
                        else:
                            w1, w2 = sc_pair(pnext, _CHB - 1, d3n, pnext)
                        w1.wait()
                        w2.wait()

                    if bi == 0:
                        @pl.when(kk + 1 < _NCH)
                        def _():
                            pltpu.async_copy(src_h.at[wid, kk + 1],
                                             srcb[pnext], ixs[pnext])
                            pltpu.async_copy(dst_h.at[wid, kk + 1],
                                             dstb[pnext], ixs[pnext])

                    if bi == _CHB - 2:
                        @pl.when(b + 2 < _NBW)
                        def _():
                            pltpu.make_async_copy(
                                src_h.at[wid, kk + 1], srcb[pnext],
                                ixs[pnext]).wait()
                            pltpu.make_async_copy(
                                dst_h.at[wid, kk + 1], dstb[pnext],
                                ixs[pnext]).wait()

                    @pl.when(b + 2 < _NBW)
                    def _():
                        if bi < _CHB - 2:
                            g_xl(p, bi + 2, d3n).start()
                            g_xr(p, bi + 2, d2).start()
                        else:
                            g_xl(pnext, bi - (_CHB - 2), d3n).start()
                            g_xr(pnext, bi - (_CHB - 2), d2).start()

            return carry

        lax.fori_loop(0, _NCH // 2, body, 0)

        w1, w2 = sc_pair(1, _CHB - 1, (_CHB - 1) % 3, (1 + _CHB - 1) % 2)
        w1.wait()
        w2.wait()

        plsc.subcore_barrier()
        pltpu.sync_copy(s_sh.at[pl.ds(r0, _RPT)],
                        s_out.at[cid, pl.ds(r0, _RPT)])

        @pl.when(sid == 0)
        def _():
            pltpu.sync_copy(den_sh, den_out.at[cid])

    return k(xl, xr, src_r, dst_r, att, zero_nc, zero_np)


def _pre_body(x_r, wl_r, bl_r, wr_r, br_r, xl_o, xr_o):
    h = x_r[...]
    pad = jnp.zeros((_NPAD - _N, _C), jnp.float32)
    xl_o[...] = jnp.concatenate(
        [jnp.dot(h, wl_r[...], preferred_element_type=jnp.float32) + bl_r[...],
         pad], axis=0)
    xr_o[...] = jnp.concatenate(
        [jnp.dot(h, wr_r[...], preferred_element_type=jnp.float32) + br_r[...],
         pad], axis=0)


def _combine(s_r, den_r, bias_r, wlin_r, blin_r, gamma_r, beta_r, do_relu):
    s3 = s_r[...]
    s = s3[0, :_N] + s3[1, :_N]
    out = s / (den_r[...] + 1e-16) + bias_r[...]
    t = out + jnp.dot(out, wlin_r[...],
                      preferred_element_type=jnp.float32) + blin_r[...]
    mean = jnp.mean(t, axis=0, keepdims=True)
    var = jnp.mean((t - mean) ** 2, axis=0, keepdims=True)
    hh = (t - mean) * lax.rsqrt(var + _BN_EPS) * gamma_r[...] + beta_r[...]
    if do_relu:
        hh = jnp.maximum(hh, 0.0)
    return hh


def _mid_body(s_r, den_r, bias_r, wlin_r, blin_r, gamma_r, beta_r,
              wl_r, bl_r, wr_r, br_r, xl_o, xr_o):
    hh = _combine(s_r, den_r, bias_r, wlin_r, blin_r, gamma_r, beta_r, True)
    pad = jnp.zeros((_NPAD - _N, _C), jnp.float32)
    xl_o[...] = jnp.concatenate(
        [jnp.dot(hh, wl_r[...],
                 preferred_element_type=jnp.float32) + bl_r[...], pad], axis=0)
    xr_o[...] = jnp.concatenate(
        [jnp.dot(hh, wr_r[...],
                 preferred_element_type=jnp.float32) + br_r[...], pad], axis=0)


def _fin_body(s_r, den_r, bias_r, wlin_r, blin_r, gamma_r, beta_r, h_o):
    h_o[...] = _combine(s_r, den_r, bias_r, wlin_r, blin_r, gamma_r, beta_r,
                        False)


def kernel(x, edge_index, params):
    npd = _E2 - _E
    src_r = jnp.concatenate(
        [edge_index[0],
         jnp.zeros((npd,), jnp.int32)]).reshape(_NW, _NCH, _CHB * _BLK)
    dst_r = jnp.concatenate(
        [edge_index[1],
         jnp.full((npd,), _NPAD - 1, jnp.int32)]).reshape(_NW, _NCH, _CHB,
                                                          _BLK)
    zero_nc = jnp.zeros((_NPAD, _C), jnp.float32)
    zero_np = jnp.zeros((_NPAD,), jnp.float32)
    p0, p1 = params

    ncp = jax.ShapeDtypeStruct((_NPAD, _C), jnp.float32)
    nc = jax.ShapeDtypeStruct((_N, _C), jnp.float32)

    xl, xr = pl.pallas_call(
        _pre_body, out_shape=[ncp, ncp],
    )(x, p0['Wl'], p0['bl'].reshape(1, _C), p0['Wr'], p0['br'].reshape(1, _C))

    s2, den2 = _sc_edge_call(xl, xr, src_r, dst_r, p0['att'],
                             zero_nc, zero_np)
    den = (den2[0, :_N] + den2[1, :_N]).reshape(_N, 1)

    xl2, xr2 = pl.pallas_call(
        _mid_body, out_shape=[ncp, ncp],
    )(s2, den, p0['bias'].reshape(1, _C), p0['Wlin'],
      p0['blin'].reshape(1, _C), p0['gamma'].reshape(1, _C),
      p0['beta'].reshape(1, _C), p1['Wl'], p1['bl'].reshape(1, _C),
      p1['Wr'], p1['br'].reshape(1, _C))

    s2b, den2b = _sc_edge_call(xl2, xr2, src_r, dst_r, p1['att'],
                               zero_nc, zero_np)
    denb = (den2b[0, :_N] + den2b[1, :_N]).reshape(_N, 1)

    h = pl.pallas_call(
        _fin_body, out_shape=nc,
    )(s2b, denb, p1['bias'].reshape(1, _C), p1['Wlin'],
      p1['blin'].reshape(1, _C), p1['gamma'].reshape(1, _C),
      p1['beta'].reshape(1, _C))

    return h


# speedup vs baseline: 9.2289x; 4.0390x over previous
import functools

import jax
import jax.numpy as jnp
from jax import lax
from jax.experimental import pallas as pl
from jax.experimental.pallas import tpu as pltpu
from jax.experimental.pallas import tpu_sc as plsc

_N = 10000
_E = 320000
_C = 128
_NEG = 0.2
_BN_EPS = 1e-5
_NPAD = 10112
_BLK = 64
_NW = 32
_CHB = 9
_NCH = 18
_NBW = _NCH * _CHB
_NB = _NW * _NBW
_E2 = _NB * _BLK
_RPT = _NPAD // 16


def _sc_edge_call(xl, xr, src_r, dst_r, att, zero_nc, zero_np):
    mesh = plsc.VectorSubcoreMesh(core_axis_name="c", subcore_axis_name="s")

    @functools.partial(
        pl.kernel,
        out_type=[
            jax.ShapeDtypeStruct((2, _NPAD, _C), jnp.float32),
            jax.ShapeDtypeStruct((2, _NPAD), jnp.float32),
        ],
        mesh=mesh,
        compiler_params=pltpu.CompilerParams(needs_layout_passes=False),
        scratch_types=[
            [pltpu.VMEM((_CHB * _BLK,), jnp.int32)] * 2,
            [pltpu.VMEM((_CHB, _BLK), jnp.int32)] * 2,
            [pltpu.VMEM((_BLK, _C), jnp.float32)] * 3,
            [pltpu.VMEM((_BLK, _C), jnp.float32)] * 2,
            [pltpu.VMEM((_BLK,), jnp.float32)] * 2,
            pltpu.VMEM((_C,), jnp.float32),
            pltpu.VMEM_SHARED((_NPAD, _C), jnp.float32),
            pltpu.VMEM_SHARED((_NPAD,), jnp.float32),
            [pltpu.SemaphoreType.DMA] * 3,
            [pltpu.SemaphoreType.DMA] * 2,
            [pltpu.SemaphoreType.DMA] * 3,
            [pltpu.SemaphoreType.DMA] * 2,
        ],
    )
    def k(xl_h, xr_h, src_h, dst_h, att_h, znc_h, znp_h, s_out, den_out,
          srcb, dstb, xlr, xrr, ex_v, att_v, s_sh, den_sh,
          gls, grs, scs, ixs):
        cid = lax.axis_index("c")
        sid = lax.axis_index("s")
        wid = cid * 16 + sid

        r0 = sid * _RPT
        pltpu.sync_copy(znc_h.at[pl.ds(r0, _RPT)], s_sh.at[pl.ds(r0, _RPT)])

        @pl.when(sid == 0)
        def _():
            pltpu.sync_copy(znp_h, den_sh)

        pltpu.sync_copy(att_h, att_v)
        pltpu.sync_copy(src_h.at[wid, 0], srcb[0])
        pltpu.sync_copy(dst_h.at[wid, 0], dstb[0])
        plsc.subcore_barrier()

        def g_xl(bp, bi, d):
            return pltpu.make_async_copy(
                xl_h.at[srcb[bp].at[pl.ds(bi * _BLK, _BLK)]], xlr[d], gls[d])

        def g_xr(bp, bi, d):
            return pltpu.make_async_copy(
                xr_h.at[dstb[bp].at[bi]], xrr[d], grs[d])

        def sc_pair(bp, bi, d, de):
            return (pltpu.make_async_copy(ex_v[de],
                                          den_sh.at[dstb[bp].at[bi]],
                                          scs[d]),
                    pltpu.make_async_copy(xlr[d], s_sh.at[dstb[bp].at[bi]],
                                          scs[d]))

        g_xl(0, 0, 0).start()
        g_xr(0, 0, 0).start()
        g_xl(0, 1, 1).start()
        g_xr(0, 1, 1).start()

        def compute(d3, d2):
            lanes = lax.iota(jnp.int32, 16)

            def group(g, carry):
                ev = lanes + g * 16

                def cstep(t, acc):
                    cvec = (t + lanes) & (_C - 1)
                    vl = plsc.load_gather(xlr[d3], [ev, cvec])
                    vr = plsc.load_gather(xrr[d2], [ev, cvec])
                    av = plsc.load_gather(att_v, [cvec])
                    v = vl + vr
                    lr = jnp.maximum(v, _NEG * v)
                    return acc + lr * av

                acc = lax.fori_loop(0, _C, cstep,
                                    jnp.zeros((16,), jnp.float32), unroll=8)
                ex16 = jnp.exp(acc)
                ex_v[d2][pl.ds(pl.multiple_of(g * 16, 16), 16)] = ex16

                def sstep(t, carry2):
                    cvec = (t + lanes) & (_C - 1)
                    vl = plsc.load_gather(xlr[d3], [ev, cvec])
                    plsc.store_scatter(xlr[d3], [ev, cvec], vl * ex16)
                    return carry2

                lax.fori_loop(0, _C, sstep, 0, unroll=8)
                return carry

            lax.fori_loop(0, _BLK // 16, group, 0)

        def body(ii, carry):
            for p in range(2):
                kk = ii * 2 + p
                pnext = (p + 1) % 2

                for bi in range(_CHB):
                    b = kk * _CHB + bi
                    d3 = bi % 3
                    d2 = (p + bi) % 2
                    g_xl(p, bi, d3).wait()
                    g_xr(p, bi, d2).wait()
                    compute(d3, d2)
                    e1, e2 = sc_pair(p, bi, d3, d2)
                    e1.start(add=True)
                    e2.start(add=True)

                    d3n = (bi + 2) % 3

                    @pl.when(b >= 1)
                    def _():
                        if bi >= 1:
                            w1, w2 = sc_pair(p, bi - 1, d3n, (p + bi - 1) % 2)
                        else:
                            w1, w2 = sc_pair(pnext, _CHB - 1, d3n, pnext)
                        w1.wait()
                        w2.wait()

                    if bi == 0:
                        @pl.when(kk + 1 < _NCH)
                        def _():
                            pltpu.async_copy(src_h.at[wid, kk + 1],
                                             srcb[pnext], ixs[pnext])
                            pltpu.async_copy(dst_h.at[wid, kk + 1],
                                             dstb[pnext], ixs[pnext])

                    if bi == _CHB - 2:
                        @pl.when(b + 2 < _NBW)
                        def _():
                            pltpu.make_async_copy(
                                src_h.at[wid, kk + 1], srcb[pnext],
                                ixs[pnext]).wait()
                            pltpu.make_async_copy(
                                dst_h.at[wid, kk + 1], dstb[pnext],
                                ixs[pnext]).wait()

                    @pl.when(b + 2 < _NBW)
                    def _():
                        if bi < _CHB - 2:
                            g_xl(p, bi + 2, d3n).start()
                            g_xr(p, bi + 2, d2).start()
                        else:
                            g_xl(pnext, bi - (_CHB - 2), d3n).start()
                            g_xr(pnext, bi - (_CHB - 2), d2).start()

            return carry

        lax.fori_loop(0, _NCH // 2, body, 0)

        w1, w2 = sc_pair(1, _CHB - 1, (_CHB - 1) % 3, (1 + _CHB - 1) % 2)
        w1.wait()
        w2.wait()

        plsc.subcore_barrier()
        pltpu.sync_copy(s_sh.at[pl.ds(r0, _RPT)],
                        s_out.at[cid, pl.ds(r0, _RPT)])

        @pl.when(sid == 0)
        def _():
            pltpu.sync_copy(den_sh, den_out.at[cid])

    return k(xl, xr, src_r, dst_r, att, zero_nc, zero_np)


def _pre_body(x_r, wl_r, bl_r, wr_r, br_r, xl_o, xr_o):
    h = x_r[...]
    pad = jnp.zeros((_NPAD - _N, _C), jnp.float32)
    xl_o[...] = jnp.concatenate(
        [jnp.dot(h, wl_r[...], preferred_element_type=jnp.float32) + bl_r[...],
         pad], axis=0)
    xr_o[...] = jnp.concatenate(
        [jnp.dot(h, wr_r[...], preferred_element_type=jnp.float32) + br_r[...],
         pad], axis=0)


def _combine(s_r, den_r, bias_r, wlin_r, blin_r, gamma_r, beta_r, do_relu):
    s3 = s_r[...]
    s = s3[0, :_N] + s3[1, :_N]
    out = s / (den_r[...] + 1e-16) + bias_r[...]
    t = out + jnp.dot(out, wlin_r[...],
                      preferred_element_type=jnp.float32) + blin_r[...]
    mean = jnp.mean(t, axis=0, keepdims=True)
    var = jnp.mean((t - mean) ** 2, axis=0, keepdims=True)
    hh = (t - mean) * lax.rsqrt(var + _BN_EPS) * gamma_r[...] + beta_r[...]
    if do_relu:
        hh = jnp.maximum(hh, 0.0)
    return hh


def _mid_body(s_r, den_r, bias_r, wlin_r, blin_r, gamma_r, beta_r,
              wl_r, bl_r, wr_r, br_r, xl_o, xr_o):
    hh = _combine(s_r, den_r, bias_r, wlin_r, blin_r, gamma_r, beta_r, True)
    pad = jnp.zeros((_NPAD - _N, _C), jnp.float32)
    xl_o[...] = jnp.concatenate(
        [jnp.dot(hh, wl_r[...],
                 preferred_element_type=jnp.float32) + bl_r[...], pad], axis=0)
    xr_o[...] = jnp.concatenate(
        [jnp.dot(hh, wr_r[...],
                 preferred_element_type=jnp.float32) + br_r[...], pad], axis=0)


def _fin_body(s_r, den_r, bias_r, wlin_r, blin_r, gamma_r, beta_r, h_o):
    h_o[...] = _combine(s_r, den_r, bias_r, wlin_r, blin_r, gamma_r, beta_r,
                        False)


def kernel(x, edge_index, params):
    npd = _E2 - _E
    src_r = jnp.concatenate(
        [edge_index[0],
         jnp.zeros((npd,), jnp.int32)]).reshape(_NW, _NCH, _CHB * _BLK)
    dst_r = jnp.concatenate(
        [edge_index[1],
         jnp.full((npd,), _NPAD - 1, jnp.int32)]).reshape(_NW, _NCH, _CHB,
                                                          _BLK)
    zero_nc = jnp.zeros((_NPAD, _C), jnp.float32)
    zero_np = jnp.zeros((_NPAD,), jnp.float32)
    p0, p1 = params

    ncp = jax.ShapeDtypeStruct((_NPAD, _C), jnp.float32)
    nc = jax.ShapeDtypeStruct((_N, _C), jnp.float32)

    xl, xr = pl.pallas_call(
        _pre_body, out_shape=[ncp, ncp],
    )(x, p0['Wl'], p0['bl'].reshape(1, _C), p0['Wr'], p0['br'].reshape(1, _C))

    s2, den2 = _sc_edge_call(xl, xr, src_r, dst_r, p0['att'],
                             zero_nc, zero_np)
    den = (den2[0, :_N] + den2[1, :_N]).reshape(_N, 1)

    xl2, xr2 = pl.pallas_call(
        _mid_body, out_shape=[ncp, ncp],
    )(s2, den, p0['bias'].reshape(1, _C), p0['Wlin'],
      p0['blin'].reshape(1, _C), p0['gamma'].reshape(1, _C),
      p0['beta'].reshape(1, _C), p1['Wl'], p1['bl'].reshape(1, _C),
      p1['Wr'], p1['br'].reshape(1, _C))

    s2b, den2b = _sc_edge_call(xl2, xr2, src_r, dst_r, p1['att'],
                               zero_nc, zero_np)
    denb = (den2b[0, :_N] + den2b[1, :_N]).reshape(_N, 1)

    h = pl.pallas_call(
        _fin_body, out_shape=nc,
    )(s2b, denb, p1['bias'].reshape(1, _C), p1['Wlin'],
      p1['blin'].reshape(1, _C), p1['gamma'].reshape(1, _C),
      p1['beta'].reshape(1, _C))

    return h


# speedup vs baseline: 9.3204x; 1.0099x over previous
import functools

import jax
import jax.numpy as jnp
from jax import lax
from jax.experimental import pallas as pl
from jax.experimental.pallas import tpu as pltpu
from jax.experimental.pallas import tpu_sc as plsc

_N = 10000
_E = 320000
_C = 128
_NEG = 0.2
_BN_EPS = 1e-5
_NPAD = 10112
_BLK = 64
_NW = 32
_CHB = 9
_NCH = 18
_NBW = _NCH * _CHB
_NB = _NW * _NBW
_E2 = _NB * _BLK
_RPT = _NPAD // 16


def _sc_edge_call(xl, xr, src_r, dst_r, att, zero_nc, zero_np):
    mesh = plsc.VectorSubcoreMesh(core_axis_name="c", subcore_axis_name="s")

    @functools.partial(
        pl.kernel,
        out_type=[
            jax.ShapeDtypeStruct((2, _NPAD, _C), jnp.float32),
            jax.ShapeDtypeStruct((2, _NPAD), jnp.float32),
        ],
        mesh=mesh,
        compiler_params=pltpu.CompilerParams(needs_layout_passes=False),
        scratch_types=[
            [pltpu.VMEM((_CHB * _BLK,), jnp.int32)] * 2,
            [pltpu.VMEM((_CHB, _BLK), jnp.int32)] * 2,
            [pltpu.VMEM((_BLK, _C), jnp.float32)] * 3,
            [pltpu.VMEM((_BLK, _C), jnp.float32)] * 2,
            [pltpu.VMEM((_BLK,), jnp.float32)] * 2,
            pltpu.VMEM((_C,), jnp.float32),
            pltpu.VMEM_SHARED((_NPAD, _C), jnp.float32),
            pltpu.VMEM_SHARED((_NPAD,), jnp.float32),
            [pltpu.SemaphoreType.DMA] * 3,
            [pltpu.SemaphoreType.DMA] * 2,
            [pltpu.SemaphoreType.DMA] * 3,
            [pltpu.SemaphoreType.DMA] * 2,
        ],
    )
    def k(xl_h, xr_h, src_h, dst_h, att_h, znc_h, znp_h, s_out, den_out,
          srcb, dstb, xlr, xrr, ex_v, att_v, s_sh, den_sh,
          gls, grs, scs, ixs):
        cid = lax.axis_index("c")
        sid = lax.axis_index("s")
        wid = cid * 16 + sid
        r0 = sid * _RPT

        pltpu.sync_copy(src_h.at[wid, 0], srcb[0])
        pltpu.sync_copy(dst_h.at[wid, 0], dstb[0])

        def g_xl(bp, bi, d):
            return pltpu.make_async_copy(
                xl_h.at[srcb[bp].at[pl.ds(bi * _BLK, _BLK)]], xlr[d], gls[d])

        def g_xr(bp, bi, d):
            return pltpu.make_async_copy(
                xr_h.at[dstb[bp].at[bi]], xrr[d], grs[d])

        def sc_pair(bp, bi, d, de):
            return (pltpu.make_async_copy(ex_v[de],
                                          den_sh.at[dstb[bp].at[bi]],
                                          scs[d]),
                    pltpu.make_async_copy(xlr[d], s_sh.at[dstb[bp].at[bi]],
                                          scs[d]))

        g_xl(0, 0, 0).start()
        g_xr(0, 0, 0).start()
        g_xl(0, 1, 1).start()
        g_xr(0, 1, 1).start()

        pltpu.sync_copy(znc_h.at[pl.ds(r0, _RPT)], s_sh.at[pl.ds(r0, _RPT)])

        @pl.when(sid == 0)
        def _():
            pltpu.sync_copy(znp_h, den_sh)

        pltpu.sync_copy(att_h, att_v)
        plsc.subcore_barrier()

        def compute(d3, d2):
            lanes = lax.iota(jnp.int32, 16)

            def group(g, carry):
                ev = lanes + g * 16

                def cstep(t, acc):
                    cvec = (t + lanes) & (_C - 1)
                    vl = plsc.load_gather(xlr[d3], [ev, cvec])
                    vr = plsc.load_gather(xrr[d2], [ev, cvec])
                    av = plsc.load_gather(att_v, [cvec])
                    v = vl + vr
                    lr = jnp.maximum(v, _NEG * v)
                    return acc + lr * av

                acc = lax.fori_loop(0, _C, cstep,
                                    jnp.zeros((16,), jnp.float32), unroll=8)
                ex16 = jnp.exp(acc)
                ex_v[d2][pl.ds(pl.multiple_of(g * 16, 16), 16)] = ex16

                def sstep(t, carry2):
                    cvec = (t + lanes) & (_C - 1)
                    vl = plsc.load_gather(xlr[d3], [ev, cvec])
                    plsc.store_scatter(xlr[d3], [ev, cvec], vl * ex16)
                    return carry2

                lax.fori_loop(0, _C, sstep, 0, unroll=8)
                return carry

            lax.fori_loop(0, _BLK // 16, group, 0)

        def body(ii, carry):
            for p in range(2):
                kk = ii * 2 + p
                pnext = (p + 1) % 2

                for bi in range(_CHB):
                    b = kk * _CHB + bi
                    d3 = bi % 3
                    d2 = (p + bi) % 2
                    g_xl(p, bi, d3).wait()
                    g_xr(p, bi, d2).wait()
                    compute(d3, d2)
                    e1, e2 = sc_pair(p, bi, d3, d2)
                    e1.start(add=True)
                    e2.start(add=True)

                    d3n = (bi + 2) % 3

                    @pl.when(b >= 1)
                    def _():
                        if bi >= 1:
                            w1, w2 = sc_pair(p, bi - 1, d3n, (p + bi - 1) % 2)
                        else:
                            w1, w2 = sc_pair(pnext, _CHB - 1, d3n, pnext)
                        w1.wait()
                        w2.wait()

                    if bi == 0:
                        @pl.when(kk + 1 < _NCH)
                        def _():
                            pltpu.async_copy(src_h.at[wid, kk + 1],
                                             srcb[pnext], ixs[pnext])
                            pltpu.async_copy(dst_h.at[wid, kk + 1],
                                             dstb[pnext], ixs[pnext])

                    if bi == _CHB - 2:
                        @pl.when(b + 2 < _NBW)
                        def _():
                            pltpu.make_async_copy(
                                src_h.at[wid, kk + 1], srcb[pnext],
                                ixs[pnext]).wait()
                            pltpu.make_async_copy(
                                dst_h.at[wid, kk + 1], dstb[pnext],
                                ixs[pnext]).wait()

                    @pl.when(b + 2 < _NBW)
                    def _():
                        if bi < _CHB - 2:
                            g_xl(p, bi + 2, d3n).start()
                            g_xr(p, bi + 2, d2).start()
                        else:
                            g_xl(pnext, bi - (_CHB - 2), d3n).start()
                            g_xr(pnext, bi - (_CHB - 2), d2).start()

            return carry

        lax.fori_loop(0, _NCH // 2, body, 0)

        w1, w2 = sc_pair(1, _CHB - 1, (_CHB - 1) % 3, (1 + _CHB - 1) % 2)
        w1.wait()
        w2.wait()

        plsc.subcore_barrier()
        pltpu.sync_copy(s_sh.at[pl.ds(r0, _RPT)],
                        s_out.at[cid, pl.ds(r0, _RPT)])

        @pl.when(sid == 0)
        def _():
            pltpu.sync_copy(den_sh, den_out.at[cid])

    return k(xl, xr, src_r, dst_r, att, zero_nc, zero_np)


def _pre_body(x_r, wl_r, bl_r, wr_r, br_r, xl_o, xr_o):
    h = x_r[...]
    pad = jnp.zeros((_NPAD - _N, _C), jnp.float32)
    xl_o[...] = jnp.concatenate(
        [jnp.dot(h, wl_r[...], preferred_element_type=jnp.float32) + bl_r[...],
         pad], axis=0)
    xr_o[...] = jnp.concatenate(
        [jnp.dot(h, wr_r[...], preferred_element_type=jnp.float32) + br_r[...],
         pad], axis=0)


def _combine(s_r, den_r, bias_r, wlin_r, blin_r, gamma_r, beta_r, do_relu):
    s3 = s_r[...]
    s = s3[0, :_N] + s3[1, :_N]
    out = s / (den_r[...] + 1e-16) + bias_r[...]
    t = out + jnp.dot(out, wlin_r[...],
                      preferred_element_type=jnp.float32) + blin_r[...]
    mean = jnp.mean(t, axis=0, keepdims=True)
    var = jnp.mean((t - mean) ** 2, axis=0, keepdims=True)
    hh = (t - mean) * lax.rsqrt(var + _BN_EPS) * gamma_r[...] + beta_r[...]
    if do_relu:
        hh = jnp.maximum(hh, 0.0)
    return hh


def _mid_body(s_r, den_r, bias_r, wlin_r, blin_r, gamma_r, beta_r,
              wl_r, bl_r, wr_r, br_r, xl_o, xr_o):
    hh = _combine(s_r, den_r, bias_r, wlin_r, blin_r, gamma_r, beta_r, True)
    pad = jnp.zeros((_NPAD - _N, _C), jnp.float32)
    xl_o[...] = jnp.concatenate(
        [jnp.dot(hh, wl_r[...],
                 preferred_element_type=jnp.float32) + bl_r[...], pad], axis=0)
    xr_o[...] = jnp.concatenate(
        [jnp.dot(hh, wr_r[...],
                 preferred_element_type=jnp.float32) + br_r[...], pad], axis=0)


def _fin_body(s_r, den_r, bias_r, wlin_r, blin_r, gamma_r, beta_r, h_o):
    h_o[...] = _combine(s_r, den_r, bias_r, wlin_r, blin_r, gamma_r, beta_r,
                        False)


def kernel(x, edge_index, params):
    npd = _E2 - _E
    src_r = jnp.concatenate(
        [edge_index[0],
         jnp.zeros((npd,), jnp.int32)]).reshape(_NW, _NCH, _CHB * _BLK)
    dst_r = jnp.concatenate(
        [edge_index[1],
         jnp.full((npd,), _NPAD - 1, jnp.int32)]).reshape(_NW, _NCH, _CHB,
                                                          _BLK)
    zero_nc = jnp.zeros((_NPAD, _C), jnp.float32)
    zero_np = jnp.zeros((_NPAD,), jnp.float32)
    p0, p1 = params

    ncp = jax.ShapeDtypeStruct((_NPAD, _C), jnp.float32)
    nc = jax.ShapeDtypeStruct((_N, _C), jnp.float32)

    xl, xr = pl.pallas_call(
        _pre_body, out_shape=[ncp, ncp],
    )(x, p0['Wl'], p0['bl'].reshape(1, _C), p0['Wr'], p0['br'].reshape(1, _C))

    s2, den2 = _sc_edge_call(xl, xr, src_r, dst_r, p0['att'],
                             zero_nc, zero_np)
    den = (den2[0, :_N] + den2[1, :_N]).reshape(_N, 1)

    xl2, xr2 = pl.pallas_call(
        _mid_body, out_shape=[ncp, ncp],
    )(s2, den, p0['bias'].reshape(1, _C), p0['Wlin'],
      p0['blin'].reshape(1, _C), p0['gamma'].reshape(1, _C),
      p0['beta'].reshape(1, _C), p1['Wl'], p1['bl'].reshape(1, _C),
      p1['Wr'], p1['br'].reshape(1, _C))

    s2b, den2b = _sc_edge_call(xl2, xr2, src_r, dst_r, p1['att'],
                               zero_nc, zero_np)
    denb = (den2b[0, :_N] + den2b[1, :_N]).reshape(_N, 1)

    h = pl.pallas_call(
        _fin_body, out_shape=nc,
    )(s2b, denb, p1['bias'].reshape(1, _C), p1['Wlin'],
      p1['blin'].reshape(1, _C), p1['gamma'].reshape(1, _C),
      p1['beta'].reshape(1, _C))

    return h
